# baseline (speedup 1.0000x reference)
import jax
import jax.numpy as jnp
from jax import lax
from jax.experimental import pallas as pl
from jax.experimental.pallas import tpu as pltpu
from jax.experimental.pallas import tpu_sc as plsc

N = 10000
E = 320000
D = 128
D_OUT = 64
NC = 2
NS = 16
NW = NC * NS
EPW = E // NW
CHUNK = 80
NCHUNK = EPW // CHUNK
RPT = 624
TAIL0 = NS * RPT
TAILN = N - TAIL0

_mesh = plsc.VectorSubcoreMesh(
    core_axis_name="c", subcore_axis_name="s", num_cores=NC, num_subcores=NS)


def _fill(ref, rows, cols, vec):
    def row(i, _):
        def col(j, __):
            ref[i, pl.ds(j * 16, 16)] = vec
            return __
        return lax.fori_loop(0, cols // 16, col, _)
    lax.fori_loop(0, rows, row, 0)


def _zero_spmem_rows(acc, zbuf, s):
    r0 = s * RPT
    def z(k, _):
        pltpu.sync_copy(zbuf, acc.at[pl.ds(r0 + k * 16, 16)])
        return _
    lax.fori_loop(0, RPT // 16, z, 0)

    @pl.when(s == 0)
    def _tail():
        pltpu.sync_copy(zbuf, acc.at[pl.ds(TAIL0, TAILN)])


def _copy_out(acc, out, c, s):
    r0 = s * RPT
    pltpu.sync_copy(acc.at[pl.ds(r0, RPT)], out.at[pl.ds(c * N + r0, RPT)])

    @pl.when(s == 0)
    def _tail_out():
        pltpu.sync_copy(acc.at[pl.ds(TAIL0, TAILN)],
                        out.at[pl.ds(c * N + TAIL0, TAILN)])


NPAIR = NCHUNK // 2
_NIB = 4
_NRB = 2
NQUAD = NCHUNK // _NIB


@pl.kernel(out_type=[jax.ShapeDtypeStruct((NC * N, D), jnp.float32),
                     jax.ShapeDtypeStruct((NC * N, D), jnp.float32)],
           mesh=_mesh,
           scratch_types=(
               [pltpu.VMEM((CHUNK,), jnp.int32) for _ in range(2 * _NIB)] +
               [pltpu.VMEM((CHUNK, D), jnp.float32) for _ in range(_NRB)] +
               [pltpu.VMEM((16, D), jnp.float32)] +
               [pltpu.VMEM_SHARED((N, D), jnp.float32)] +
               [pltpu.SemaphoreType.DMA for _ in range(_NIB + _NRB)]))
def _sc_deg_sum(table, src, dst, out_deg, out_sum, *bufs):
    SRC = bufs[0:_NIB]
    DST = bufs[_NIB:2 * _NIB]
    ROW = bufs[2 * _NIB:2 * _NIB + _NRB]
    zbuf = bufs[2 * _NIB + _NRB]
    acc = bufs[2 * _NIB + _NRB + 1]
    SI = bufs[2 * _NIB + _NRB + 2:2 * _NIB + _NRB + 2 + _NIB]
    SG = bufs[2 * _NIB + _NRB + 2 + _NIB:]
    c = lax.axis_index("c")
    s = lax.axis_index("s")
    g = c * NS + s
    ebase = g * EPW

    def iad(t, j):
        pltpu.async_copy(dst.at[pl.ds(ebase + t * CHUNK, CHUNK)], DST[j], SI[j])

    def iad_wait(t, j):
        pltpu.make_async_copy(
            dst.at[pl.ds(ebase + t * CHUNK, CHUNK)], DST[j], SI[j]).wait()

    iad(0, 0)
    _fill(zbuf, 16, D, jnp.zeros((16,), jnp.float32))
    _fill(ROW[0], CHUNK, D, jnp.ones((16,), jnp.float32))
    _zero_spmem_rows(acc, zbuf, s)
    plsc.subcore_barrier()

    def dstep(t, _):
        a = 2 * t
        iad(a + 1, 1)
        iad_wait(a, 0)
        pltpu.sync_copy(ROW[0], acc.at[DST[0]], add=True)
        iad(a + 2, 0)
        iad_wait(a + 1, 1)
        pltpu.sync_copy(ROW[0], acc.at[DST[1]], add=True)
        return _
    lax.fori_loop(0, NPAIR, dstep, 0)
    iad_wait(NCHUNK - 1, 0)
    pltpu.sync_copy(ROW[0], acc.at[DST[0]], add=True)
    plsc.subcore_barrier()
    _copy_out(acc, out_deg, c, s)

    def ia(t, j):
        base = ebase + t * CHUNK
        pltpu.async_copy(src.at[pl.ds(base, CHUNK)], SRC[j], SI[j])
        pltpu.async_copy(dst.at[pl.ds(base, CHUNK)], DST[j], SI[j])

    def ia_wait(t, j):
        base = ebase + t * CHUNK
        pltpu.make_async_copy(src.at[pl.ds(base, CHUNK)], SRC[j], SI[j]).wait()
        pltpu.make_async_copy(dst.at[pl.ds(base, CHUNK)], DST[j], SI[j]).wait()

    def ga(j, p):
        pltpu.async_copy(table.at[SRC[j]], ROW[p], SG[p])

    def ga_wait(j, p):
        pltpu.make_async_copy(table.at[SRC[j]], ROW[p], SG[p]).wait()

    for j in range(_NIB):
        ia(j, j)
    ia_wait(0, 0)
    ga(0, 0)
    ia_wait(1, 1)
    ga(1, 1)
    _zero_spmem_rows(acc, zbuf, s)
    plsc.subcore_barrier()

    def _process(k, j, p):
        ga_wait(j, p)
        pltpu.sync_copy(ROW[p], acc.at[DST[j]], add=True)
        jn = (j + 2) % _NIB

        @pl.when(k + 2 < NCHUNK)
        def _g():
            ia_wait(k + 2, jn)
            ga(jn, p)

        @pl.when(k + 4 < NCHUNK)
        def _pf():
            ia(k + 4, j)

    def step(t, _):
        for j in range(_NIB):
            _process(_NIB * t + j, j, j % _NRB)
        return _
    lax.fori_loop(0, NQUAD, step, 0)
    _process(NCHUNK - 1, (NCHUNK - 1) % _NIB, (NCHUNK - 1) % _NRB)
    plsc.subcore_barrier()
    _copy_out(acc, out_sum, c, s)


@pl.kernel(out_type=jax.ShapeDtypeStruct((NC * N, D), jnp.float32), mesh=_mesh,
           scratch_types=(
               [pltpu.VMEM((CHUNK,), jnp.int32) for _ in range(2 * _NIB)] +
               [pltpu.VMEM((CHUNK, D), jnp.float32) for _ in range(_NRB)] +
               [pltpu.VMEM((16, D), jnp.float32)] +
               [pltpu.VMEM_SHARED((N, D), jnp.float32)] +
               [pltpu.SemaphoreType.DMA for _ in range(_NIB + _NRB)]))
def _sc_sum(table, src, dst, out_sum, *bufs):
    SRC = bufs[0:_NIB]
    DST = bufs[_NIB:2 * _NIB]
    ROW = bufs[2 * _NIB:2 * _NIB + _NRB]
    zbuf = bufs[2 * _NIB + _NRB]
    acc = bufs[2 * _NIB + _NRB + 1]
    SI = bufs[2 * _NIB + _NRB + 2:2 * _NIB + _NRB + 2 + _NIB]
    SG = bufs[2 * _NIB + _NRB + 2 + _NIB:]
    c = lax.axis_index("c")
    s = lax.axis_index("s")
    g = c * NS + s
    ebase = g * EPW

    def ia(t, j):
        base = ebase + t * CHUNK
        pltpu.async_copy(src.at[pl.ds(base, CHUNK)], SRC[j], SI[j])
        pltpu.async_copy(dst.at[pl.ds(base, CHUNK)], DST[j], SI[j])

    def ia_wait(t, j):
        base = ebase + t * CHUNK
        pltpu.make_async_copy(src.at[pl.ds(base, CHUNK)], SRC[j], SI[j]).wait()
        pltpu.make_async_copy(dst.at[pl.ds(base, CHUNK)], DST[j], SI[j]).wait()

    def ga(j, p):
        pltpu.async_copy(table.at[SRC[j]], ROW[p], SG[p])

    def ga_wait(j, p):
        pltpu.make_async_copy(table.at[SRC[j]], ROW[p], SG[p]).wait()

    for j in range(_NIB):
        ia(j, j)
    ia_wait(0, 0)
    ga(0, 0)
    ia_wait(1, 1)
    ga(1, 1)
    _fill(zbuf, 16, D, jnp.zeros((16,), jnp.float32))
    _zero_spmem_rows(acc, zbuf, s)
    plsc.subcore_barrier()

    def _process(k, j, p):
        ga_wait(j, p)
        pltpu.sync_copy(ROW[p], acc.at[DST[j]], add=True)
        jn = (j + 2) % _NIB

        @pl.when(k + 2 < NCHUNK)
        def _g():
            ia_wait(k + 2, jn)
            ga(jn, p)

        @pl.when(k + 4 < NCHUNK)
        def _pf():
            ia(k + 4, j)

    def step(t, _):
        for j in range(_NIB):
            _process(_NIB * t + j, j, j % _NRB)
        return _
    lax.fori_loop(0, NQUAD, step, 0)
    _process(NCHUNK - 1, (NCHUNK - 1) % _NIB, (NCHUNK - 1) % _NRB)
    plsc.subcore_barrier()
    _copy_out(acc, out_sum, c, s)


R = 1000


def _tc1_body(p0, p1, d0, d1, w1, h1):
    deg = jnp.maximum((d0[...] + d1[...])[:, :1], 1.0)
    agg = (p0[...] + p1[...]) / deg
    h = lax.dot_general(agg, w1[...], (((1,), (1,)), ((), ())),
                        preferred_element_type=jnp.float32)
    h1[...] = jnp.maximum(h, 0.0)


def _tc2_body(q0, q1, d0, d1, w2, wo, bo, out, hid):
    deg = jnp.maximum((d0[...] + d1[...])[:, :1], 1.0)
    agg = (q0[...] + q1[...]) / deg
    h = lax.dot_general(agg, w2[...], (((1,), (1,)), ((), ())),
                        preferred_element_type=jnp.float32)
    hid[...] = h
    logits = lax.dot_general(h, wo[...], (((1,), (1,)), ((), ())),
                             preferred_element_type=jnp.float32) + bo[...]
    m = jnp.max(logits, axis=1, keepdims=True)
    ex = jnp.exp(logits - m)
    lse = jnp.log(jnp.sum(ex, axis=1, keepdims=True)) + m
    out[...] = logits - lse


def _tc1(ps, pd, W1):
    nb = N // R
    return pl.pallas_call(
        _tc1_body,
        grid=(nb,),
        in_specs=[
            pl.BlockSpec((R, D), lambda i: (i, 0)),
            pl.BlockSpec((R, D), lambda i: (i + nb, 0)),
            pl.BlockSpec((R, D), lambda i: (i, 0)),
            pl.BlockSpec((R, D), lambda i: (i + nb, 0)),
            pl.BlockSpec((D, D), lambda i: (0, 0)),
        ],
        out_specs=pl.BlockSpec((R, D), lambda i: (i, 0)),
        out_shape=jax.ShapeDtypeStruct((N, D), jnp.float32),
    )(ps, ps, pd, pd, W1)


def _tc2(qs, pd, W2, Wo, bo2):
    nb = N // R
    return pl.pallas_call(
        _tc2_body,
        grid=(nb,),
        in_specs=[
            pl.BlockSpec((R, D), lambda i: (i, 0)),
            pl.BlockSpec((R, D), lambda i: (i + nb, 0)),
            pl.BlockSpec((R, D), lambda i: (i, 0)),
            pl.BlockSpec((R, D), lambda i: (i + nb, 0)),
            pl.BlockSpec((D, D), lambda i: (0, 0)),
            pl.BlockSpec((D_OUT, D), lambda i: (0, 0)),
            pl.BlockSpec((1, D_OUT), lambda i: (0, 0)),
        ],
        out_specs=[
            pl.BlockSpec((R, D_OUT), lambda i: (i, 0)),
            pl.BlockSpec((R, D), lambda i: (i, 0)),
        ],
        out_shape=[
            jax.ShapeDtypeStruct((N, D_OUT), jnp.float32),
            jax.ShapeDtypeStruct((N, D), jnp.float32),
        ],
    )(qs, qs, pd, pd, W2, Wo, bo2)


def kernel(x, edge_index, W1, W2, Wo, bo):
    src = edge_index[0]
    dst = edge_index[1]
    pd, ps = _sc_deg_sum(x, src, dst)
    h1 = _tc1(ps, pd, W1)
    qs = _sc_sum(h1, src, dst)
    outputs, hidden = _tc2(qs, pd, W2, Wo, bo.reshape(1, D_OUT))
    return (outputs, hidden)


# speedup vs baseline: 1.0010x; 1.0010x over previous
import jax
import jax.numpy as jnp
from jax import lax
from jax.experimental import pallas as pl
from jax.experimental.pallas import tpu as pltpu
from jax.experimental.pallas import tpu_sc as plsc

N = 10000
E = 320000
D = 128
D_OUT = 64
NC = 2
NS = 16
NW = NC * NS
EPW = E // NW
CHUNK = 80
NCHUNK = EPW // CHUNK
RPT = 624
TAIL0 = NS * RPT
TAILN = N - TAIL0

_mesh = plsc.VectorSubcoreMesh(
    core_axis_name="c", subcore_axis_name="s", num_cores=NC, num_subcores=NS)


def _fill(ref, rows, cols, vec):
    def row(i, _):
        def col(j, __):
            ref[i, pl.ds(j * 16, 16)] = vec
            return __
        return lax.fori_loop(0, cols // 16, col, _)
    lax.fori_loop(0, rows, row, 0)


def _zero_spmem_rows(acc, zbuf, s):
    r0 = s * RPT
    def z(k, _):
        pltpu.sync_copy(zbuf, acc.at[pl.ds(r0 + k * 16, 16)])
        return _
    lax.fori_loop(0, RPT // 16, z, 0)

    @pl.when(s == 0)
    def _tail():
        pltpu.sync_copy(zbuf, acc.at[pl.ds(TAIL0, TAILN)])


def _copy_out(acc, out, c, s):
    r0 = s * RPT
    pltpu.sync_copy(acc.at[pl.ds(r0, RPT)], out.at[pl.ds(c * N + r0, RPT)])

    @pl.when(s == 0)
    def _tail_out():
        pltpu.sync_copy(acc.at[pl.ds(TAIL0, TAILN)],
                        out.at[pl.ds(c * N + TAIL0, TAILN)])


_NIB = 4
_NRB = 2
NQUAD = NCHUNK // _NIB


@pl.kernel(out_type=[jax.ShapeDtypeStruct((NC * N, D), jnp.float32),
                     jax.ShapeDtypeStruct((NC * N, D), jnp.float32)],
           mesh=_mesh,
           scratch_types=(
               [pltpu.VMEM((CHUNK,), jnp.int32) for _ in range(2 * _NIB)] +
               [pltpu.VMEM((CHUNK, D), jnp.float32) for _ in range(_NRB)] +
               [pltpu.VMEM((16, D), jnp.float32)] +
               [pltpu.VMEM_SHARED((N, D), jnp.float32)] +
               [pltpu.SemaphoreType.DMA for _ in range(_NIB + _NRB)]))
def _sc_deg_sum(table, src, dst, out_deg, out_sum, *bufs):
    SRC = bufs[0:_NIB]
    DST = bufs[_NIB:2 * _NIB]
    ROW = bufs[2 * _NIB:2 * _NIB + _NRB]
    zbuf = bufs[2 * _NIB + _NRB]
    acc = bufs[2 * _NIB + _NRB + 1]
    SI = bufs[2 * _NIB + _NRB + 2:2 * _NIB + _NRB + 2 + _NIB]
    SG = bufs[2 * _NIB + _NRB + 2 + _NIB:]
    c = lax.axis_index("c")
    s = lax.axis_index("s")
    g = c * NS + s
    ebase = g * EPW

    def iad(t, j):
        pltpu.async_copy(dst.at[pl.ds(ebase + t * CHUNK, CHUNK)], DST[j], SI[j])

    def iad_wait(t, j):
        pltpu.make_async_copy(
            dst.at[pl.ds(ebase + t * CHUNK, CHUNK)], DST[j], SI[j]).wait()

    for j in range(_NIB):
        iad(j, j)
    _fill(zbuf, 16, D, jnp.zeros((16,), jnp.float32))
    _fill(ROW[0], CHUNK, D, jnp.ones((16,), jnp.float32))
    _zero_spmem_rows(acc, zbuf, s)
    plsc.subcore_barrier()

    def _dprocess(k, j):
        iad_wait(k, j)
        pltpu.sync_copy(ROW[0], acc.at[DST[j]], add=True)

        @pl.when(k + _NIB < NCHUNK)
        def _pf():
            iad(k + _NIB, j)

    def dstep(t, _):
        for j in range(_NIB):
            _dprocess(_NIB * t + j, j)
        return _
    lax.fori_loop(0, NQUAD, dstep, 0)
    _dprocess(NCHUNK - 1, (NCHUNK - 1) % _NIB)
    plsc.subcore_barrier()
    _copy_out(acc, out_deg, c, s)

    def ia(t, j):
        base = ebase + t * CHUNK
        pltpu.async_copy(src.at[pl.ds(base, CHUNK)], SRC[j], SI[j])
        pltpu.async_copy(dst.at[pl.ds(base, CHUNK)], DST[j], SI[j])

    def ia_wait(t, j):
        base = ebase + t * CHUNK
        pltpu.make_async_copy(src.at[pl.ds(base, CHUNK)], SRC[j], SI[j]).wait()
        pltpu.make_async_copy(dst.at[pl.ds(base, CHUNK)], DST[j], SI[j]).wait()

    def ga(j, p):
        pltpu.async_copy(table.at[SRC[j]], ROW[p], SG[p])

    def ga_wait(j, p):
        pltpu.make_async_copy(table.at[SRC[j]], ROW[p], SG[p]).wait()

    for j in range(_NIB):
        ia(j, j)
    ia_wait(0, 0)
    ga(0, 0)
    ia_wait(1, 1)
    ga(1, 1)
    _zero_spmem_rows(acc, zbuf, s)
    plsc.subcore_barrier()

    def _process(k, j, p):
        ga_wait(j, p)
        pltpu.sync_copy(ROW[p], acc.at[DST[j]], add=True)
        jn = (j + 2) % _NIB

        @pl.when(k + 2 < NCHUNK)
        def _g():
            ia_wait(k + 2, jn)
            ga(jn, p)

        @pl.when(k + 4 < NCHUNK)
        def _pf():
            ia(k + 4, j)

    def step(t, _):
        for j in range(_NIB):
            _process(_NIB * t + j, j, j % _NRB)
        return _
    lax.fori_loop(0, NQUAD, step, 0)
    _process(NCHUNK - 1, (NCHUNK - 1) % _NIB, (NCHUNK - 1) % _NRB)
    plsc.subcore_barrier()
    _copy_out(acc, out_sum, c, s)


@pl.kernel(out_type=jax.ShapeDtypeStruct((NC * N, D), jnp.float32), mesh=_mesh,
           scratch_types=(
               [pltpu.VMEM((CHUNK,), jnp.int32) for _ in range(2 * _NIB)] +
               [pltpu.VMEM((CHUNK, D), jnp.float32) for _ in range(_NRB)] +
               [pltpu.VMEM((16, D), jnp.float32)] +
               [pltpu.VMEM_SHARED((N, D), jnp.float32)] +
               [pltpu.SemaphoreType.DMA for _ in range(_NIB + _NRB)]))
def _sc_sum(table, src, dst, out_sum, *bufs):
    SRC = bufs[0:_NIB]
    DST = bufs[_NIB:2 * _NIB]
    ROW = bufs[2 * _NIB:2 * _NIB + _NRB]
    zbuf = bufs[2 * _NIB + _NRB]
    acc = bufs[2 * _NIB + _NRB + 1]
    SI = bufs[2 * _NIB + _NRB + 2:2 * _NIB + _NRB + 2 + _NIB]
    SG = bufs[2 * _NIB + _NRB + 2 + _NIB:]
    c = lax.axis_index("c")
    s = lax.axis_index("s")
    g = c * NS + s
    ebase = g * EPW

    def ia(t, j):
        base = ebase + t * CHUNK
        pltpu.async_copy(src.at[pl.ds(base, CHUNK)], SRC[j], SI[j])
        pltpu.async_copy(dst.at[pl.ds(base, CHUNK)], DST[j], SI[j])

    def ia_wait(t, j):
        base = ebase + t * CHUNK
        pltpu.make_async_copy(src.at[pl.ds(base, CHUNK)], SRC[j], SI[j]).wait()
        pltpu.make_async_copy(dst.at[pl.ds(base, CHUNK)], DST[j], SI[j]).wait()

    def ga(j, p):
        pltpu.async_copy(table.at[SRC[j]], ROW[p], SG[p])

    def ga_wait(j, p):
        pltpu.make_async_copy(table.at[SRC[j]], ROW[p], SG[p]).wait()

    for j in range(_NIB):
        ia(j, j)
    ia_wait(0, 0)
    ga(0, 0)
    ia_wait(1, 1)
    ga(1, 1)
    _fill(zbuf, 16, D, jnp.zeros((16,), jnp.float32))
    _zero_spmem_rows(acc, zbuf, s)
    plsc.subcore_barrier()

    def _process(k, j, p):
        ga_wait(j, p)
        pltpu.sync_copy(ROW[p], acc.at[DST[j]], add=True)
        jn = (j + 2) % _NIB

        @pl.when(k + 2 < NCHUNK)
        def _g():
            ia_wait(k + 2, jn)
            ga(jn, p)

        @pl.when(k + 4 < NCHUNK)
        def _pf():
            ia(k + 4, j)

    def step(t, _):
        for j in range(_NIB):
            _process(_NIB * t + j, j, j % _NRB)
        return _
    lax.fori_loop(0, NQUAD, step, 0)
    _process(NCHUNK - 1, (NCHUNK - 1) % _NIB, (NCHUNK - 1) % _NRB)
    plsc.subcore_barrier()
    _copy_out(acc, out_sum, c, s)


R = 1000


def _tc1_body(p0, p1, d0, d1, w1, h1):
    deg = jnp.maximum((d0[...] + d1[...])[:, :1], 1.0)
    agg = (p0[...] + p1[...]) / deg
    h = lax.dot_general(agg, w1[...], (((1,), (1,)), ((), ())),
                        preferred_element_type=jnp.float32)
    h1[...] = jnp.maximum(h, 0.0)


def _tc2_body(q0, q1, d0, d1, w2, wo, bo, out, hid):
    deg = jnp.maximum((d0[...] + d1[...])[:, :1], 1.0)
    agg = (q0[...] + q1[...]) / deg
    h = lax.dot_general(agg, w2[...], (((1,), (1,)), ((), ())),
                        preferred_element_type=jnp.float32)
    hid[...] = h
    logits = lax.dot_general(h, wo[...], (((1,), (1,)), ((), ())),
                             preferred_element_type=jnp.float32) + bo[...]
    m = jnp.max(logits, axis=1, keepdims=True)
    ex = jnp.exp(logits - m)
    lse = jnp.log(jnp.sum(ex, axis=1, keepdims=True)) + m
    out[...] = logits - lse


def _tc1(ps, pd, W1):
    nb = N // R
    return pl.pallas_call(
        _tc1_body,
        grid=(nb,),
        in_specs=[
            pl.BlockSpec((R, D), lambda i: (i, 0)),
            pl.BlockSpec((R, D), lambda i: (i + nb, 0)),
            pl.BlockSpec((R, D), lambda i: (i, 0)),
            pl.BlockSpec((R, D), lambda i: (i + nb, 0)),
            pl.BlockSpec((D, D), lambda i: (0, 0)),
        ],
        out_specs=pl.BlockSpec((R, D), lambda i: (i, 0)),
        out_shape=jax.ShapeDtypeStruct((N, D), jnp.float32),
    )(ps, ps, pd, pd, W1)


def _tc2(qs, pd, W2, Wo, bo2):
    nb = N // R
    return pl.pallas_call(
        _tc2_body,
        grid=(nb,),
        in_specs=[
            pl.BlockSpec((R, D), lambda i: (i, 0)),
            pl.BlockSpec((R, D), lambda i: (i + nb, 0)),
            pl.BlockSpec((R, D), lambda i: (i, 0)),
            pl.BlockSpec((R, D), lambda i: (i + nb, 0)),
            pl.BlockSpec((D, D), lambda i: (0, 0)),
            pl.BlockSpec((D_OUT, D), lambda i: (0, 0)),
            pl.BlockSpec((1, D_OUT), lambda i: (0, 0)),
        ],
        out_specs=[
            pl.BlockSpec((R, D_OUT), lambda i: (i, 0)),
            pl.BlockSpec((R, D), lambda i: (i, 0)),
        ],
        out_shape=[
            jax.ShapeDtypeStruct((N, D_OUT), jnp.float32),
            jax.ShapeDtypeStruct((N, D), jnp.float32),
        ],
    )(qs, qs, pd, pd, W2, Wo, bo2)


def kernel(x, edge_index, W1, W2, Wo, bo):
    src = edge_index[0]
    dst = edge_index[1]
    pd, ps = _sc_deg_sum(x, src, dst)
    h1 = _tc1(ps, pd, W1)
    qs = _sc_sum(h1, src, dst)
    outputs, hidden = _tc2(qs, pd, W2, Wo, bo.reshape(1, D_OUT))
    return (outputs, hidden)
